# baseline (speedup 1.0000x reference)
import functools

import jax
import jax.numpy as jnp
from jax import lax
from jax.experimental import pallas as pl
from jax.experimental.pallas import tpu as pltpu
from jax.experimental.pallas import tpu_sc as plsc

N = 10000
E = 320000
D = 128
NC, NS = 2, 16
NW = NC * NS
CHUNK = 128
CPW = 80
EPAD = NW * CPW * CHUNK
DUMP = 112
ACC = N + DUMP
RPS = ACC // NS

_MESH = plsc.VectorSubcoreMesh(core_axis_name="c", subcore_axis_name="s")


CPS = EPAD // (NS * CHUNK)


@functools.partial(
    pl.kernel,
    out_type=jax.ShapeDtypeStruct((NC * NS * ACC,), jnp.int32),
    mesh=_MESH,
    scratch_types=[
        pltpu.VMEM((CPS * CHUNK,), jnp.int32),
        pltpu.VMEM((ACC,), jnp.int32),
    ],
    compiler_params=pltpu.CompilerParams(needs_layout_passes=False,
                                         skip_device_barrier=True),
)
def _deg_kernel(keys, zeros_hbm, hists, key_v, hist_v):
    cid = lax.axis_index("c")
    sid = lax.axis_index("s")
    pltpu.sync_copy(zeros_hbm, hist_v)
    pltpu.sync_copy(keys.at[cid, sid], key_v)

    def body(j, carry):
        for q in range(4):
            kv = key_v[pl.ds((j * 4 + q) * 16, 16)]
            cnt, last = plsc.scan_count(kv)
            plsc.addupdate_scatter(hist_v, [kv], cnt, mask=last)
        return carry

    lax.fori_loop(0, CPS * CHUNK // 64, body, 0)
    off = (cid * NS + sid) * ACC
    pltpu.sync_copy(hist_v, hists.at[pl.ds(off, ACC)])


NWIN = 4
WCH = CPW // NWIN


HC = CHUNK // 2


@functools.partial(
    pl.kernel,
    out_type=jax.ShapeDtypeStruct((NC, ACC, D), jnp.float32),
    mesh=_MESH,
    scratch_types=[
        pltpu.VMEM((2, WCH, 2, CHUNK), jnp.int32),
        pltpu.VMEM((2, CHUNK, D), jnp.float32),
        pltpu.VMEM_SHARED((ACC, D), jnp.float32),
        pltpu.SemaphoreType.DMA,
        pltpu.SemaphoreType.DMA,
        pltpu.SemaphoreType.DMA,
        pltpu.SemaphoreType.DMA,
        pltpu.SemaphoreType.DMA,
        pltpu.SemaphoreType.DMA,
    ],
    compiler_params=pltpu.CompilerParams(skip_device_barrier=True),
)
def _agg_kernel(sd_hbm, t_hbm, zeros_hbm, out,
                idx_v, rows_v, acc, ga0, gb0, ga1, gb1, ws0, ws1):
    cid = lax.axis_index("c")
    sid = lax.axis_index("s")
    wid = sid * NC + cid
    r0 = sid * RPS
    wsems = (ws0, ws1)
    gsa = (ga0, ga1)
    gsb = (gb0, gb1)

    def gather(wb, kk, rb):
        pltpu.async_copy(t_hbm.at[idx_v.at[wb, kk, 0]], rows_v.at[rb], gsa[rb])

    def gather_wait(wb, kk, rb):
        pltpu.make_async_copy(t_hbm.at[idx_v.at[wb, kk, 0]],
                              rows_v.at[rb], gsa[rb]).wait()

    zdesc = pltpu.async_copy(zeros_hbm.at[pl.ds(r0, RPS)],
                             acc.at[pl.ds(r0, RPS)], gsb[0])
    pltpu.sync_copy(sd_hbm.at[wid, pl.ds(0, WCH)], idx_v.at[0])
    zdesc.wait()
    plsc.subcore_barrier()

    for w in range(NWIN):
        b = w % 2
        if w > 0:
            pltpu.make_async_copy(sd_hbm.at[wid, pl.ds(w * WCH, WCH)],
                                  idx_v.at[b], wsems[b]).wait()
        if w + 1 < NWIN:
            nb = (w + 1) % 2
            pltpu.async_copy(sd_hbm.at[wid, pl.ds((w + 1) * WCH, WCH)],
                             idx_v.at[nb], wsems[nb])
        gather(b, 0, 0)
        gather(b, 1, 1)

        @pl.loop(0, WCH, step=2)
        def _ring(k):
            gather_wait(b, k, 0)
            pltpu.sync_copy(rows_v.at[0], acc.at[idx_v.at[b, k, 1]], add=True)

            @pl.when(k + 2 < WCH)
            def _pf0():
                gather(b, k + 2, 0)

            gather_wait(b, k + 1, 1)
            pltpu.sync_copy(rows_v.at[1], acc.at[idx_v.at[b, k + 1, 1]], add=True)

            @pl.when(k + 3 < WCH)
            def _pf1():
                gather(b, k + 3, 1)

    plsc.subcore_barrier()
    pltpu.sync_copy(acc.at[pl.ds(r0, RPS)], out.at[cid, pl.ds(r0, RPS)])


def _tc_first_body(h_ref, feat_ref, w_ref, t_ref, ns_ref, nd_ref):
    do = jnp.sum(h_ref[:N, 0:NS], axis=1, keepdims=True).astype(jnp.float32)
    di = jnp.sum(h_ref[:N, NS:2 * NS], axis=1, keepdims=True).astype(jnp.float32)
    ns = lax.rsqrt(jnp.maximum(do, 1.0))
    nd = lax.rsqrt(jnp.maximum(di, 1.0))
    ns_ref[...] = ns
    nd_ref[...] = nd
    t_ref[...] = jnp.dot(feat_ref[...] * ns, w_ref[...],
                         preferred_element_type=jnp.float32)


def _tc_mid_body(p_ref, hp_ref, nd_ref, ns_ref, b_ref, g_ref, be_ref, w_ref,
                 h_ref, t_ref):
    h = (p_ref[0, :N] + p_ref[1, :N]) * nd_ref[...] + b_ref[...] + hp_ref[...]
    mu = jnp.mean(h, axis=0, keepdims=True)
    var = jnp.mean((h - mu) ** 2, axis=0, keepdims=True)
    h = (h - mu) / jnp.sqrt(var + 1e-5) * g_ref[...] + be_ref[...]
    h = jnp.maximum(h, 0.0)
    h_ref[...] = h
    t_ref[...] = jnp.dot(h * ns_ref[...], w_ref[...],
                         preferred_element_type=jnp.float32)


def _tc_final_body(p_ref, hp_ref, nd_ref, b_ref, out_ref):
    out_ref[...] = ((p_ref[0, :N] + p_ref[1, :N]) * nd_ref[...]
                    + b_ref[...] + hp_ref[...])


_tc_first = pl.pallas_call(
    _tc_first_body,
    out_shape=(jax.ShapeDtypeStruct((N, D), jnp.float32),
               jax.ShapeDtypeStruct((N, 1), jnp.float32),
               jax.ShapeDtypeStruct((N, 1), jnp.float32)),
)

_tc_mid = pl.pallas_call(
    _tc_mid_body,
    out_shape=(jax.ShapeDtypeStruct((N, D), jnp.float32),
               jax.ShapeDtypeStruct((N, D), jnp.float32)),
)

_tc_final = pl.pallas_call(
    _tc_final_body,
    out_shape=jax.ShapeDtypeStruct((N, D), jnp.float32),
)


def kernel(graph, feat, W1, b1, W2, b2, W3, b3, g1, be1, g2, be2):
    src = graph[0].astype(jnp.int32)
    dst = graph[1].astype(jnp.int32)
    pad = EPAD - E
    pad_iota = jnp.arange(pad, dtype=jnp.int32)
    src_agg = jnp.concatenate([src, pad_iota % N]).reshape(NW, CPW, CHUNK)
    dst_pad = jnp.concatenate([dst, N + pad_iota % DUMP]).reshape(NW, CPW, CHUNK)
    sd = jnp.stack([src_agg, dst_pad], axis=2)
    src_deg = jnp.concatenate([src, N + pad_iota % DUMP])

    zerosD = jnp.zeros((ACC, D), jnp.float32)
    zeros1 = jnp.zeros((ACC,), jnp.int32)

    dst_deg = jnp.concatenate([dst, N + pad_iota % DUMP])
    keys = jnp.stack([src_deg.reshape(NS, CPS * CHUNK),
                      dst_deg.reshape(NS, CPS * CHUNK)])
    hists = _deg_kernel(keys, zeros1)
    hmat = hists.reshape(NC * NS, ACC).T

    b1r, b2r, b3r = b1.reshape(1, D), b2.reshape(1, D), b3.reshape(1, D)
    g1r, be1r = g1.reshape(1, D), be1.reshape(1, D)
    g2r, be2r = g2.reshape(1, D), be2.reshape(1, D)

    t1, ns, nd = _tc_first(hmat, feat, W1)
    p1 = _agg_kernel(sd, t1, zerosD)
    h1, t2 = _tc_mid(p1, feat, nd, ns, b1r, g1r, be1r, W2)
    p2 = _agg_kernel(sd, t2, zerosD)
    h2, t3 = _tc_mid(p2, h1, nd, ns, b2r, g2r, be2r, W3)
    p3 = _agg_kernel(sd, t3, zerosD)
    return _tc_final(p3, h2, nd, b3r)


# speedup vs baseline: 1.0012x; 1.0012x over previous
import functools

import jax
import jax.numpy as jnp
from jax import lax
from jax.experimental import pallas as pl
from jax.experimental.pallas import tpu as pltpu
from jax.experimental.pallas import tpu_sc as plsc

N = 10000
E = 320000
D = 128
NC, NS = 2, 16
NW = NC * NS
CHUNK = 128
CPW = 80
EPAD = NW * CPW * CHUNK
DUMP = 112
ACC = N + DUMP
RPS = ACC // NS

_MESH = plsc.VectorSubcoreMesh(core_axis_name="c", subcore_axis_name="s")


CPS = EPAD // (NS * CHUNK)


@functools.partial(
    pl.kernel,
    out_type=jax.ShapeDtypeStruct((NC * NS * ACC,), jnp.int32),
    mesh=_MESH,
    scratch_types=[
        pltpu.VMEM((CPS * CHUNK,), jnp.int32),
        pltpu.VMEM((ACC,), jnp.int32),
    ],
    compiler_params=pltpu.CompilerParams(needs_layout_passes=False,
                                         skip_device_barrier=True),
)
def _deg_kernel(keys, zeros_hbm, hists, key_v, hist_v):
    cid = lax.axis_index("c")
    sid = lax.axis_index("s")
    pltpu.sync_copy(zeros_hbm, hist_v)
    pltpu.sync_copy(keys.at[cid, sid], key_v)

    def body(j, carry):
        for q in range(4):
            kv = key_v[pl.ds((j * 4 + q) * 16, 16)]
            cnt, last = plsc.scan_count(kv)
            plsc.addupdate_scatter(hist_v, [kv], cnt, mask=last)
        return carry

    lax.fori_loop(0, CPS * CHUNK // 64, body, 0)
    off = (cid * NS + sid) * ACC
    pltpu.sync_copy(hist_v, hists.at[pl.ds(off, ACC)])


NWIN = 4
WCH = CPW // NWIN


@functools.partial(
    pl.kernel,
    out_type=jax.ShapeDtypeStruct((NC, ACC, D), jnp.float32),
    mesh=_MESH,
    scratch_types=[
        pltpu.VMEM((2, WCH, 2, CHUNK), jnp.int32),
        pltpu.VMEM((2, CHUNK, D), jnp.float32),
        pltpu.VMEM_SHARED((ACC, D), jnp.float32),
        pltpu.SemaphoreType.DMA,
        pltpu.SemaphoreType.DMA,
        pltpu.SemaphoreType.DMA,
        pltpu.SemaphoreType.DMA,
        pltpu.SemaphoreType.DMA,
        pltpu.SemaphoreType.DMA,
    ],
    compiler_params=pltpu.CompilerParams(skip_device_barrier=True),
)
def _agg_kernel(sd_hbm, t_hbm, zeros_hbm, out,
                idx_v, rows_v, acc, ga0, gb0, ga1, gb1, ws0, ws1):
    cid = lax.axis_index("c")
    sid = lax.axis_index("s")
    wid = sid * NC + cid
    r0 = sid * RPS
    wsems = (ws0, ws1)
    gsa = (ga0, ga1)
    zsem = gb0
    del gb1

    def gather(wb, kk, rb):
        pltpu.async_copy(t_hbm.at[idx_v.at[wb, kk, 0]], rows_v.at[rb], gsa[rb])

    def gather_wait(wb, kk, rb):
        pltpu.make_async_copy(t_hbm.at[idx_v.at[wb, kk, 0]],
                              rows_v.at[rb], gsa[rb]).wait()

    zdesc = pltpu.async_copy(zeros_hbm.at[pl.ds(r0, RPS)],
                             acc.at[pl.ds(r0, RPS)], zsem)
    pltpu.sync_copy(sd_hbm.at[wid, pl.ds(0, WCH)], idx_v.at[0])
    zdesc.wait()
    plsc.subcore_barrier()

    for w in range(NWIN):
        b = w % 2
        if w > 0:
            pltpu.make_async_copy(sd_hbm.at[wid, pl.ds(w * WCH, WCH)],
                                  idx_v.at[b], wsems[b]).wait()
        if w + 1 < NWIN:
            nb = (w + 1) % 2
            pltpu.async_copy(sd_hbm.at[wid, pl.ds((w + 1) * WCH, WCH)],
                             idx_v.at[nb], wsems[nb])
        gather(b, 0, 0)
        gather(b, 1, 1)

        @pl.loop(0, WCH, step=2)
        def _ring(k):
            gather_wait(b, k, 0)
            pltpu.sync_copy(rows_v.at[0], acc.at[idx_v.at[b, k, 1]], add=True)

            @pl.when(k + 2 < WCH)
            def _pf0():
                gather(b, k + 2, 0)

            gather_wait(b, k + 1, 1)
            pltpu.sync_copy(rows_v.at[1], acc.at[idx_v.at[b, k + 1, 1]], add=True)

            @pl.when(k + 3 < WCH)
            def _pf1():
                gather(b, k + 3, 1)

    plsc.subcore_barrier()
    pltpu.sync_copy(acc.at[pl.ds(r0, RPS)], out.at[cid, pl.ds(r0, RPS)])


def _tc_first_body(h_ref, feat_ref, w_ref, t_ref, ns_ref, nd_ref):
    do = jnp.sum(h_ref[:N, 0:NS], axis=1, keepdims=True).astype(jnp.float32)
    di = jnp.sum(h_ref[:N, NS:2 * NS], axis=1, keepdims=True).astype(jnp.float32)
    ns = lax.rsqrt(jnp.maximum(do, 1.0))
    nd = lax.rsqrt(jnp.maximum(di, 1.0))
    ns_ref[...] = ns
    nd_ref[...] = nd
    t_ref[...] = jnp.dot(feat_ref[...] * ns, w_ref[...],
                         preferred_element_type=jnp.float32)


def _tc_mid_body(p_ref, hp_ref, nd_ref, ns_ref, b_ref, g_ref, be_ref, w_ref,
                 h_ref, t_ref):
    h = (p_ref[0, :N] + p_ref[1, :N]) * nd_ref[...] + b_ref[...] + hp_ref[...]
    mu = jnp.mean(h, axis=0, keepdims=True)
    var = jnp.mean((h - mu) ** 2, axis=0, keepdims=True)
    h = (h - mu) / jnp.sqrt(var + 1e-5) * g_ref[...] + be_ref[...]
    h = jnp.maximum(h, 0.0)
    h_ref[...] = h
    t_ref[...] = jnp.dot(h * ns_ref[...], w_ref[...],
                         preferred_element_type=jnp.float32)


def _tc_final_body(p_ref, hp_ref, nd_ref, b_ref, out_ref):
    out_ref[...] = ((p_ref[0, :N] + p_ref[1, :N]) * nd_ref[...]
                    + b_ref[...] + hp_ref[...])


_tc_first = pl.pallas_call(
    _tc_first_body,
    out_shape=(jax.ShapeDtypeStruct((N, D), jnp.float32),
               jax.ShapeDtypeStruct((N, 1), jnp.float32),
               jax.ShapeDtypeStruct((N, 1), jnp.float32)),
)

_tc_mid = pl.pallas_call(
    _tc_mid_body,
    out_shape=(jax.ShapeDtypeStruct((N, D), jnp.float32),
               jax.ShapeDtypeStruct((N, D), jnp.float32)),
)

_tc_final = pl.pallas_call(
    _tc_final_body,
    out_shape=jax.ShapeDtypeStruct((N, D), jnp.float32),
)


def kernel(graph, feat, W1, b1, W2, b2, W3, b3, g1, be1, g2, be2):
    src = graph[0].astype(jnp.int32)
    dst = graph[1].astype(jnp.int32)
    pad = EPAD - E
    pad_iota = jnp.arange(pad, dtype=jnp.int32)
    src_agg = jnp.concatenate([src, pad_iota % N]).reshape(NW, CPW, CHUNK)
    dst_pad = jnp.concatenate([dst, N + pad_iota % DUMP]).reshape(NW, CPW, CHUNK)
    sd = jnp.stack([src_agg, dst_pad], axis=2)
    src_deg = jnp.concatenate([src, N + pad_iota % DUMP])

    zerosD = jnp.zeros((ACC, D), jnp.float32)
    zeros1 = jnp.zeros((ACC,), jnp.int32)

    dst_deg = jnp.concatenate([dst, N + pad_iota % DUMP])
    keys = jnp.stack([src_deg.reshape(NS, CPS * CHUNK),
                      dst_deg.reshape(NS, CPS * CHUNK)])
    hists = _deg_kernel(keys, zeros1)
    hmat = hists.reshape(NC * NS, ACC).T

    b1r, b2r, b3r = b1.reshape(1, D), b2.reshape(1, D), b3.reshape(1, D)
    g1r, be1r = g1.reshape(1, D), be1.reshape(1, D)
    g2r, be2r = g2.reshape(1, D), be2.reshape(1, D)

    t1, ns, nd = _tc_first(hmat, feat, W1)
    p1 = _agg_kernel(sd, t1, zerosD)
    h1, t2 = _tc_mid(p1, feat, nd, ns, b1r, g1r, be1r, W2)
    p2 = _agg_kernel(sd, t2, zerosD)
    h2, t3 = _tc_mid(p2, h1, nd, ns, b2r, g2r, be2r, W3)
    p3 = _agg_kernel(sd, t3, zerosD)
    return _tc_final(p3, h2, nd, b3r)
